# Initial kernel scaffold; baseline (speedup 1.0000x reference)
import jax
import jax.numpy as jnp
from jax.experimental import pallas as pl


def kernel(heatmap, rot_cos, rot_sin, center, center_z, dim):
    raise NotImplementedError("write your pallas kernel here")


# speedup vs baseline: 3.3067x; 3.3067x over previous
import functools

import jax
import jax.numpy as jnp
from jax.experimental import pallas as pl
from jax.experimental.pallas import tpu as pltpu

K = 500
KP = 512


def _epilogue_body(data_ref, box_ref, mask_ref):
    xs = data_ref[:, 1, :] * 0.8 - 75.2
    ys = data_ref[:, 2, :] * 0.8 - 75.2
    z = data_ref[:, 3, :]
    angle = jnp.arctan2(data_ref[:, 7, :], data_ref[:, 8, :])
    box_ref[:, 0, :] = xs
    box_ref[:, 1, :] = ys
    box_ref[:, 2, :] = z
    box_ref[:, 3, :] = data_ref[:, 4, :]
    box_ref[:, 4, :] = data_ref[:, 5, :]
    box_ref[:, 5, :] = data_ref[:, 6, :]
    box_ref[:, 6, :] = angle
    box_ref[:, 7, :] = angle
    inb = (xs >= -80.0) & (xs <= 80.0) & (ys >= -80.0) & (ys <= 80.0)
    inb = inb & (z >= -10.0) & (z <= 10.0)
    mask_ref[:, :] = inb.astype(jnp.float32)


def _epilogue(data):
    B = data.shape[0]
    box, maskf = pl.pallas_call(
        _epilogue_body,
        out_shape=(
            jax.ShapeDtypeStruct((B, 8, KP), jnp.float32),
            jax.ShapeDtypeStruct((B, KP), jnp.float32),
        ),
    )(data)
    return box, maskf


def kernel(heatmap, rot_cos, rot_sin, center, center_z, dim):
    B, C, H, W = heatmap.shape
    HW = H * W
    flat = heatmap.reshape(B, C * HW)
    score, ind = jax.lax.top_k(flat, K)
    cls = (ind >> 18).astype(jnp.int32)
    hw = ind & (HW - 1)
    ysr = (hw >> 9).astype(jnp.float32)
    xsr = (hw & (W - 1)).astype(jnp.float32)

    def g(feat):
        f = feat.reshape(B, feat.shape[1], HW)
        return jnp.take_along_axis(f, hw[:, None, :], axis=2)

    cg = g(center)
    zg = g(center_z)[:, 0]
    dg = g(dim)
    sg = g(rot_sin)[:, 0]
    cosg = g(rot_cos)[:, 0]

    data = jnp.stack(
        [score, xsr + cg[:, 0], ysr + cg[:, 1], zg,
         dg[:, 0], dg[:, 1], dg[:, 2], sg, cosg,
         jnp.zeros_like(score)], axis=1)
    data = jnp.pad(data, ((0, 0), (0, 0), (0, KP - K)))
    box, maskf = _epilogue(data)
    box = jnp.transpose(box[:, :, :K], (0, 2, 1))[:, :, :7]
    return box, score, cls, maskf[:, :K] > 0.5


# speedup vs baseline: 21.4440x; 6.4850x over previous
import functools

import jax
import jax.numpy as jnp
from jax import lax
from jax.experimental import pallas as pl
from jax.experimental.pallas import tpu as pltpu
from jax.experimental.pallas import tpu_sc as plsc

B, C, H, W = 4, 10, 512, 512
HW = H * W
CHW = C * HW
K = 500
KP = 512
NSUB = 16
CHUNK = CHW // NSUB
NWIN = 8
WIN = CHUNK // NWIN
NACC = 8
GRP = WIN // (16 * NACC)
CAP1 = 256
CAP2 = 128
POOL = NSUB * CAP2
SCAP = 1024
SENT_I = 0x7FFFFFFF
L = 16


def _splat_f(x):
    return jnp.broadcast_to(x, (L,)).astype(jnp.float32)


def _splat_i(x):
    return jnp.broadcast_to(jnp.asarray(x, jnp.int32), (L,))


def _shuf(v, idx):
    return lax.gather(
        v, idx[:, None],
        lax.GatherDimensionNumbers(offset_dims=(), collapsed_slice_dims=(0,),
                                   start_index_map=(0,)),
        slice_sizes=(1,), mode=lax.GatherScatterMode.PROMISE_IN_BOUNDS)


def _iota():
    return lax.iota(jnp.int32, L)


def _tree_sum(x):
    it = _iota()
    for k in (1, 2, 4, 8):
        x = x + _shuf(x, (it + k) & (L - 1))
    return x


def _tree_max(x):
    it = _iota()
    for k in (1, 2, 4, 8):
        x = jnp.maximum(x, _shuf(x, (it + k) & (L - 1)))
    return x


def _mask01(m):
    return jnp.where(m, _splat_i(1), _splat_i(0))


def _hillis(mi):
    it = _iota()
    x = mi
    for k in (1, 2, 4, 8):
        sh = _shuf(x, jnp.maximum(it - k, 0))
        x = x + jnp.where(it >= k, sh, _splat_i(0))
    return x


def _bsearch_ge(ref, nvregs, rank):
    def count_ge(t):
        tv = _splat_f(t)

        def body(i, acc):
            v = ref[pl.ds(i * L, L)]
            return acc + _mask01(v >= tv)

        acc = lax.fori_loop(0, nvregs, body, _splat_i(0))
        return _tree_sum(acc)[0]

    def it(_, lohi):
        lo, hi = lohi
        mid = (lo + hi) * jnp.float32(0.5)
        ok = count_ge(mid) >= rank
        return jnp.where(ok, mid, lo), jnp.where(ok, hi, mid)

    lo, _ = lax.fori_loop(0, 42, it, (jnp.float32(0.0), jnp.float32(1.0)))
    return lo


def _pack_idx(mi, cum):
    it = _iota()
    packidx = it
    for l in range(L):
        pos_l = jnp.where(mi[l] > 0, cum[l] - 1, jnp.int32(-1))
        packidx = jnp.where(it == _splat_i(pos_l), _splat_i(l), packidx)
    return packidx


def _append(v, gi, t1v, cnt, dstv, dsti, cap):
    mi = _mask01(v >= t1v)
    cum = _hillis(mi)
    npos = cum[L - 1]
    packidx = _pack_idx(mi, cum)
    pv = _shuf(v, packidx)
    pg = _shuf(gi, packidx)
    at = jnp.minimum(cnt, cap - L)
    dstv[pl.ds(at, L)] = pv
    dsti[pl.ds(at, L)] = pg
    return cnt + npos


def _seal(dstv, dsti, cnt, cap, sentv, senti):
    at = jnp.minimum(cnt, cap - L)
    dstv[pl.ds(at, L)] = sentv
    dsti[pl.ds(at, L)] = senti


def _compact(srcv, srci, nvregs, t1s, dstv, dsti, cap, sentv, senti):
    t1v = _splat_f(t1s)

    def body(i, cnt):
        v = srcv[pl.ds(i * L, L)]
        tm = _tree_max(v)

        def hit(c):
            return _append(v, srci[pl.ds(i * L, L)], t1v, c, dstv, dsti, cap)

        return lax.cond(tm[0] >= t1s, hit, lambda c: c, cnt)

    cnt = lax.fori_loop(0, nvregs, body, jnp.int32(0))
    _seal(dstv, dsti, cnt, cap, sentv, senti)
    return cnt


def _sc_body(hm, rc, rs, ce, cz, dm, outs,
             winbuf, segmax, candv, candi, pubv, pubi,
             poolv, pooli, sv, si, rows,
             gia, gib0, gib1, gic0, gic1, gic2,
             gd0, gd1, gd2, gd3, gd4, gd5, gd6, gd7,
             shv, shi, sem):
    cid = lax.axis_index("c")
    sid = lax.axis_index("s")
    iota = _iota()
    sentv = _splat_f(-1.0)
    senti = _splat_i(SENT_I)

    for lb in range(2):
        b = cid * 2 + lb
        cbase = b * CHW + sid * CHUNK

        accs = tuple(sentv for _ in range(NACC))
        for win in range(NWIN):
            pltpu.sync_copy(hm.at[pl.ds(cbase + win * WIN, WIN)], winbuf)

            def gbody(g, accs):
                o = g * (L * NACC)
                return tuple(
                    jnp.maximum(accs[j], winbuf[pl.ds(o + j * L, L)])
                    for j in range(NACC))

            accs = lax.fori_loop(0, GRP, gbody, accs)
        for j in range(NACC):
            segmax[pl.ds(j * L, L)] = accs[j]

        t1s = _bsearch_ge(segmax, NACC, 64)
        t1v = _splat_f(t1s)

        cnt = jnp.int32(0)
        for win in range(NWIN):
            pltpu.sync_copy(hm.at[pl.ds(cbase + win * WIN, WIN)], winbuf)
            lbase = sid * CHUNK + win * WIN

            def cbody(i, cnt, lbase=lbase):
                v = winbuf[pl.ds(i * L, L)]
                tm = _tree_max(v)

                def hit(c):
                    gi = _splat_i(lbase + i * L) + iota
                    return _append(v, gi, t1v, c, candv, candi, CAP1)

                return lax.cond(tm[0] >= t1s, hit, lambda c: c, cnt)

            cnt = lax.fori_loop(0, WIN // L, cbody, cnt)
        _seal(candv, candi, cnt, CAP1, sentv, senti)

        t2s = _bsearch_ge(candv, CAP1 // L, 64)
        _compact(candv, candi, CAP1 // L, t2s, pubv, pubi, CAP2,
                 sentv, senti)
        pltpu.sync_copy(pubv, shv.at[pl.ds(lb * POOL + sid * CAP2, CAP2)])
        pltpu.sync_copy(pubi, shi.at[pl.ds(lb * POOL + sid * CAP2, CAP2)])
        plsc.subcore_barrier()

        pltpu.sync_copy(shv.at[pl.ds(lb * POOL, POOL)], poolv)
        pltpu.sync_copy(shi.at[pl.ds(lb * POOL, POOL)], pooli)
        tks = _bsearch_ge(poolv, POOL // L, K)

        cs = _compact(poolv, pooli, POOL // L, tks, sv, si, SCAP,
                      sentv, senti)
        nv = (cs + 15) >> 4

        for kk in range(SCAP // L // NSUB):
            j = kk * NSUB + sid

            @pl.when(j < nv)
            def _(j=j):
                vv = sv[pl.ds(j * L, L)]
                vi = si[pl.ds(j * L, L)]

                def rbody(u, r):
                    uvv = sv[pl.ds(u * L, L)]
                    uiv = si[pl.ds(u * L, L)]
                    for l in range(L):
                        av = _splat_f(uvv[l])
                        bv = _splat_i(uiv[l])
                        gt = (av > vv) | ((av == vv) & (bv < vi))
                        r = r + _mask01(gt)
                    return r

                ranks = lax.fori_loop(0, nv, rbody, _splat_i(0))
                lanev = (_splat_i(j * L) + iota) < _splat_i(cs)
                junk = _splat_i(K) + (iota & 7)
                rk = jnp.where(ranks < K, ranks, junk)
                rk = jnp.where(lanev, rk, junk)

                hw = vi & (HW - 1)
                clsv = lax.shift_right_logical(vi, 18)
                gia[...] = _splat_i(b * HW) + hw
                gib0[...] = _splat_i(b * 2 * HW) + hw
                gib1[...] = _splat_i((b * 2 + 1) * HW) + hw
                gic0[...] = _splat_i(b * 3 * HW) + hw
                gic1[...] = _splat_i((b * 3 + 1) * HW) + hw
                gic2[...] = _splat_i((b * 3 + 2) * HW) + hw
                cps = [
                    pltpu.async_copy(rc.at[gia], gd0, sem),
                    pltpu.async_copy(rs.at[gia], gd1, sem),
                    pltpu.async_copy(ce.at[gib0], gd2, sem),
                    pltpu.async_copy(ce.at[gib1], gd3, sem),
                    pltpu.async_copy(cz.at[gia], gd4, sem),
                    pltpu.async_copy(dm.at[gic0], gd5, sem),
                    pltpu.async_copy(dm.at[gic1], gd6, sem),
                    pltpu.async_copy(dm.at[gic2], gd7, sem),
                ]
                for cp in cps:
                    cp.wait()

                fields = [
                    vv,
                    clsv.astype(jnp.float32),
                    (hw & (W - 1)).astype(jnp.float32),
                    (hw >> 9).astype(jnp.float32),
                    gd2[...],
                    gd3[...],
                    gd4[...],
                    gd5[...],
                    gd6[...],
                    gd7[...],
                    gd1[...],
                    gd0[...],
                ]
                for w in range(L):
                    row = _splat_f(0.0)
                    for f, vec in enumerate(fields):
                        row = jnp.where(iota == f, _splat_f(vec[w]), row)
                    rows[pl.ds(w * L, L)] = row
                tg = _splat_i(b * KP) + rk
                wcps = []
                for w in range(L):
                    wcps.append(pltpu.async_copy(
                        rows.at[pl.ds(w * L, L)],
                        outs.at[pl.ds(tg[w] * L, L)], sem))
                for cp in wcps:
                    cp.wait()

        plsc.subcore_barrier()


def _sc_topk(hm, rc, rs, ce, cz, dm):
    mesh = plsc.VectorSubcoreMesh(core_axis_name="c", subcore_axis_name="s")
    f32, i32 = jnp.float32, jnp.int32
    kern = functools.partial(
        pl.kernel,
        out_type=jax.ShapeDtypeStruct((B * KP * L,), f32),
        mesh=mesh,
        scratch_types=[
            pltpu.VMEM((WIN,), f32),
            pltpu.VMEM((NACC * L,), f32),
            pltpu.VMEM((CAP1,), f32),
            pltpu.VMEM((CAP1,), i32),
            pltpu.VMEM((CAP2,), f32),
            pltpu.VMEM((CAP2,), i32),
            pltpu.VMEM((POOL,), f32),
            pltpu.VMEM((POOL,), i32),
            pltpu.VMEM((SCAP,), f32),
            pltpu.VMEM((SCAP,), i32),
            pltpu.VMEM((L * L,), f32),
            pltpu.VMEM((L,), i32),
            pltpu.VMEM((L,), i32),
            pltpu.VMEM((L,), i32),
            pltpu.VMEM((L,), i32),
            pltpu.VMEM((L,), i32),
            pltpu.VMEM((L,), i32),
            pltpu.VMEM((L,), f32),
            pltpu.VMEM((L,), f32),
            pltpu.VMEM((L,), f32),
            pltpu.VMEM((L,), f32),
            pltpu.VMEM((L,), f32),
            pltpu.VMEM((L,), f32),
            pltpu.VMEM((L,), f32),
            pltpu.VMEM((L,), f32),
            pltpu.VMEM_SHARED((2 * POOL,), f32),
            pltpu.VMEM_SHARED((2 * POOL,), i32),
            pltpu.SemaphoreType.DMA,
        ],
    )(_sc_body)
    return kern(hm, rc, rs, ce, cz, dm)


def _epi_body(d_ref, box_ref, aux_ref):
    xs = (d_ref[:, 2] + d_ref[:, 4]) * 0.8 - 75.2
    ys = (d_ref[:, 3] + d_ref[:, 5]) * 0.8 - 75.2
    z = d_ref[:, 6]
    angle = jnp.arctan2(d_ref[:, 10], d_ref[:, 11])
    box_ref[0, :] = xs
    box_ref[1, :] = ys
    box_ref[2, :] = z
    box_ref[3, :] = d_ref[:, 7]
    box_ref[4, :] = d_ref[:, 8]
    box_ref[5, :] = d_ref[:, 9]
    box_ref[6, :] = angle
    box_ref[7, :] = angle
    inb = (xs >= -80.0) & (xs <= 80.0) & (ys >= -80.0) & (ys <= 80.0)
    inb = inb & (z >= -10.0) & (z <= 10.0)
    aux_ref[0, :] = d_ref[:, 0]
    aux_ref[1, :] = d_ref[:, 1]
    aux_ref[2, :] = inb.astype(jnp.float32)
    aux_ref[3, :] = angle


def _epilogue(outs):
    n = B * KP
    return pl.pallas_call(
        _epi_body,
        out_shape=(
            jax.ShapeDtypeStruct((8, n), jnp.float32),
            jax.ShapeDtypeStruct((8, n), jnp.float32),
        ),
    )(outs)


def kernel(heatmap, rot_cos, rot_sin, center, center_z, dim):
    hm = heatmap.reshape(-1)
    rc = rot_cos.reshape(-1)
    rs = rot_sin.reshape(-1)
    ce = center.reshape(-1)
    cz = center_z.reshape(-1)
    dm = dim.reshape(-1)
    outs = _sc_topk(hm, rc, rs, ce, cz, dm)
    box8, aux = _epilogue(outs.reshape(B * KP, L))
    box = jnp.transpose(box8.reshape(8, B, KP), (1, 2, 0))[:, :K, :7]
    score = aux[0].reshape(B, KP)[:, :K]
    cls = aux[1].reshape(B, KP)[:, :K].astype(jnp.int32)
    mask = aux[2].reshape(B, KP)[:, :K] > 0.5
    return box, score, cls, mask


# speedup vs baseline: 47.6174x; 2.2205x over previous
import functools

import jax
import jax.numpy as jnp
from jax import lax
from jax.experimental import pallas as pl
from jax.experimental.pallas import tpu as pltpu
from jax.experimental.pallas import tpu_sc as plsc

B, C, H, W = 4, 10, 512, 512
HW = H * W
CHW = C * HW
K = 500
KP = 512
NSUB = 16
CHUNK = CHW // NSUB
NWIN = 8
WIN = CHUNK // NWIN
NACC = 8
GRP = WIN // (16 * NACC)
CAP1 = 256
CAP2 = 128
POOL = NSUB * CAP2
SCAP = 1024
SENT_I = 0x7FFFFFFF
L = 16
GL = 8 * L


def _splat_f(x):
    return jnp.broadcast_to(x, (L,)).astype(jnp.float32)


def _splat_i(x):
    return jnp.broadcast_to(jnp.asarray(x, jnp.int32), (L,))


def _shuf(v, idx):
    return lax.gather(
        v, idx[:, None],
        lax.GatherDimensionNumbers(offset_dims=(), collapsed_slice_dims=(0,),
                                   start_index_map=(0,)),
        slice_sizes=(1,), mode=lax.GatherScatterMode.PROMISE_IN_BOUNDS)


def _iota():
    return lax.iota(jnp.int32, L)


def _tree_sum(x):
    it = _iota()
    for k in (1, 2, 4, 8):
        x = x + _shuf(x, (it + k) & (L - 1))
    return x


def _tree_max(x):
    it = _iota()
    for k in (1, 2, 4, 8):
        x = jnp.maximum(x, _shuf(x, (it + k) & (L - 1)))
    return x


def _mask01(m):
    return jnp.where(m, _splat_i(1), _splat_i(0))


def _hillis(mi):
    it = _iota()
    x = mi
    for k in (1, 2, 4, 8):
        sh = _shuf(x, jnp.maximum(it - k, 0))
        x = x + jnp.where(it >= k, sh, _splat_i(0))
    return x


def _bsearch_ge(ref, nvregs, rank):
    def count_ge(t):
        tv = _splat_f(t)

        def body(i, acc):
            v = ref[pl.ds(i * L, L)]
            return acc + _mask01(v >= tv)

        acc = lax.fori_loop(0, nvregs, body, _splat_i(0))
        return _tree_sum(acc)[0]

    def it(_, lohi):
        lo, hi = lohi
        mid = (lo + hi) * jnp.float32(0.5)
        ok = count_ge(mid) >= rank
        return jnp.where(ok, mid, lo), jnp.where(ok, hi, mid)

    lo, _ = lax.fori_loop(0, 42, it, (jnp.float32(0.0), jnp.float32(1.0)))
    return lo


def _pack_idx(mi, cum):
    it = _iota()
    packidx = it
    for l in range(L):
        pos_l = jnp.where(mi[l] > 0, cum[l] - 1, jnp.int32(-1))
        packidx = jnp.where(it == _splat_i(pos_l), _splat_i(l), packidx)
    return packidx


def _append(v, gi, t1v, cnt, dstv, dsti, cap):
    mi = _mask01(v >= t1v)
    cum = _hillis(mi)
    npos = cum[L - 1]
    packidx = _pack_idx(mi, cum)
    pv = _shuf(v, packidx)
    pg = _shuf(gi, packidx)
    at = jnp.minimum(cnt, cap - L)
    dstv[pl.ds(at, L)] = pv
    dsti[pl.ds(at, L)] = pg
    return cnt + npos


def _seal(dstv, dsti, cnt, cap, sentv, senti):
    at = jnp.minimum(cnt, cap - L)
    dstv[pl.ds(at, L)] = sentv
    dsti[pl.ds(at, L)] = senti


def _compact(srcv, srci, nvregs, t1s, dstv, dsti, cap, sentv, senti):
    t1v = _splat_f(t1s)

    def body(i, cnt):
        v = srcv[pl.ds(i * L, L)]
        tm = _tree_max(v)

        def hit(c):
            return _append(v, srci[pl.ds(i * L, L)], t1v, c, dstv, dsti, cap)

        return lax.cond(tm[0] >= t1s, hit, lambda c: c, cnt)

    cnt = lax.fori_loop(0, nvregs, body, jnp.int32(0))
    _seal(dstv, dsti, cnt, cap, sentv, senti)
    return cnt


def _sc_body(hm, rc, rs, ce, cz, dm, outs,
             winbuf, gmaxbuf, segmax, candv, candi, pubv, pubi,
             poolv, pooli, sv, si, rows,
             gia, gib0, gib1, gic0, gic1, gic2,
             gd0, gd1, gd2, gd3, gd4, gd5, gd6, gd7,
             shv, shi, sem):
    cid = lax.axis_index("c")
    sid = lax.axis_index("s")
    iota = _iota()
    sentv = _splat_f(-1.0)
    senti = _splat_i(SENT_I)

    for lb in range(2):
        b = cid * 2 + lb
        cbase = b * CHW + sid * CHUNK

        def wbody1(win, accs):
            pltpu.sync_copy(hm.at[pl.ds(cbase + win * WIN, WIN)], winbuf)

            def gbody(g2, accs):
                base = g2 * (8 * GL)
                new = list(accs)
                for j in range(8):
                    gb = base + j * GL
                    gm = winbuf[pl.ds(gb, L)]
                    for t in range(1, 8):
                        gm = jnp.maximum(gm, winbuf[pl.ds(gb + t * L, L)])
                    gmaxbuf[pl.ds(win * (WIN // 8) + (g2 * 8 + j) * L, L)] = gm
                    new[j] = jnp.maximum(new[j], gm)
                return tuple(new)

            return lax.fori_loop(0, WIN // (8 * GL), gbody, accs)

        accs = lax.fori_loop(
            0, NWIN, wbody1, tuple(sentv for _ in range(NACC)))
        for j in range(NACC):
            segmax[pl.ds(j * L, L)] = accs[j]

        t1s = _bsearch_ge(segmax, NACC, 64)
        t1v = _splat_f(t1s)

        def wbody2(win, cnt):
            pltpu.sync_copy(hm.at[pl.ds(cbase + win * WIN, WIN)], winbuf)
            lbase = sid * CHUNK + win * WIN

            def cbody(g, cnt):
                gm = gmaxbuf[pl.ds(win * (WIN // 8) + g * L, L)]
                tm = _tree_max(gm)

                def ghit(c):
                    for t in range(8):
                        v = winbuf[pl.ds(g * GL + t * L, L)]
                        tv = _tree_max(v)

                        def hit(c2, v=v, t=t):
                            gi = _splat_i(lbase + g * GL + t * L) + iota
                            return _append(v, gi, t1v, c2,
                                           candv, candi, CAP1)

                        c = lax.cond(tv[0] >= t1s, hit, lambda c2: c2, c)
                    return c

                return lax.cond(tm[0] >= t1s, ghit, lambda c: c, cnt)

            return lax.fori_loop(0, WIN // GL, cbody, cnt)

        cnt = lax.fori_loop(0, NWIN, wbody2, jnp.int32(0))
        _seal(candv, candi, cnt, CAP1, sentv, senti)

        t2s = _bsearch_ge(candv, CAP1 // L, 64)
        _compact(candv, candi, CAP1 // L, t2s, pubv, pubi, CAP2,
                 sentv, senti)
        pltpu.sync_copy(pubv, shv.at[pl.ds(lb * POOL + sid * CAP2, CAP2)])
        pltpu.sync_copy(pubi, shi.at[pl.ds(lb * POOL + sid * CAP2, CAP2)])
        plsc.subcore_barrier()

        pltpu.sync_copy(shv.at[pl.ds(lb * POOL, POOL)], poolv)
        pltpu.sync_copy(shi.at[pl.ds(lb * POOL, POOL)], pooli)
        tks = _bsearch_ge(poolv, POOL // L, K)

        cs = _compact(poolv, pooli, POOL // L, tks, sv, si, SCAP,
                      sentv, senti)
        nv = (cs + 15) >> 4

        for kk in range(SCAP // L // NSUB):
            j = kk * NSUB + sid

            @pl.when(j < nv)
            def _(j=j):
                vv = sv[pl.ds(j * L, L)]
                vi = si[pl.ds(j * L, L)]

                def rbody(u, r):
                    uvv = sv[pl.ds(u * L, L)]
                    uiv = si[pl.ds(u * L, L)]
                    for l in range(L):
                        av = _splat_f(uvv[l])
                        bv = _splat_i(uiv[l])
                        gt = (av > vv) | ((av == vv) & (bv < vi))
                        r = r + _mask01(gt)
                    return r

                ranks = lax.fori_loop(0, nv, rbody, _splat_i(0))
                lanev = (_splat_i(j * L) + iota) < _splat_i(cs)
                junk = _splat_i(K) + (iota & 7)
                rk = jnp.where(ranks < K, ranks, junk)
                rk = jnp.where(lanev, rk, junk)

                hw = vi & (HW - 1)
                clsv = lax.shift_right_logical(vi, 18)
                gia[...] = _splat_i(b * HW) + hw
                gib0[...] = _splat_i(b * 2 * HW) + hw
                gib1[...] = _splat_i((b * 2 + 1) * HW) + hw
                gic0[...] = _splat_i(b * 3 * HW) + hw
                gic1[...] = _splat_i((b * 3 + 1) * HW) + hw
                gic2[...] = _splat_i((b * 3 + 2) * HW) + hw
                cps = [
                    pltpu.async_copy(rc.at[gia], gd0, sem),
                    pltpu.async_copy(rs.at[gia], gd1, sem),
                    pltpu.async_copy(ce.at[gib0], gd2, sem),
                    pltpu.async_copy(ce.at[gib1], gd3, sem),
                    pltpu.async_copy(cz.at[gia], gd4, sem),
                    pltpu.async_copy(dm.at[gic0], gd5, sem),
                    pltpu.async_copy(dm.at[gic1], gd6, sem),
                    pltpu.async_copy(dm.at[gic2], gd7, sem),
                ]
                for cp in cps:
                    cp.wait()

                fields = [
                    vv,
                    clsv.astype(jnp.float32),
                    (hw & (W - 1)).astype(jnp.float32),
                    (hw >> 9).astype(jnp.float32),
                    gd2[...],
                    gd3[...],
                    gd4[...],
                    gd5[...],
                    gd6[...],
                    gd7[...],
                    gd1[...],
                    gd0[...],
                ]
                for w in range(L):
                    row = _splat_f(0.0)
                    for f, vec in enumerate(fields):
                        row = jnp.where(iota == f, _splat_f(vec[w]), row)
                    rows[pl.ds(w * L, L)] = row
                tg = _splat_i(b * KP) + rk
                wcps = []
                for w in range(L):
                    wcps.append(pltpu.async_copy(
                        rows.at[pl.ds(w * L, L)],
                        outs.at[pl.ds(tg[w] * L, L)], sem))
                for cp in wcps:
                    cp.wait()

        plsc.subcore_barrier()


def _sc_topk(hm, rc, rs, ce, cz, dm):
    mesh = plsc.VectorSubcoreMesh(core_axis_name="c", subcore_axis_name="s")
    f32, i32 = jnp.float32, jnp.int32
    kern = functools.partial(
        pl.kernel,
        out_type=jax.ShapeDtypeStruct((B * KP * L,), f32),
        mesh=mesh,
        scratch_types=[
            pltpu.VMEM((WIN,), f32),
            pltpu.VMEM((CHUNK // 8,), f32),
            pltpu.VMEM((NACC * L,), f32),
            pltpu.VMEM((CAP1,), f32),
            pltpu.VMEM((CAP1,), i32),
            pltpu.VMEM((CAP2,), f32),
            pltpu.VMEM((CAP2,), i32),
            pltpu.VMEM((POOL,), f32),
            pltpu.VMEM((POOL,), i32),
            pltpu.VMEM((SCAP,), f32),
            pltpu.VMEM((SCAP,), i32),
            pltpu.VMEM((L * L,), f32),
            pltpu.VMEM((L,), i32),
            pltpu.VMEM((L,), i32),
            pltpu.VMEM((L,), i32),
            pltpu.VMEM((L,), i32),
            pltpu.VMEM((L,), i32),
            pltpu.VMEM((L,), i32),
            pltpu.VMEM((L,), f32),
            pltpu.VMEM((L,), f32),
            pltpu.VMEM((L,), f32),
            pltpu.VMEM((L,), f32),
            pltpu.VMEM((L,), f32),
            pltpu.VMEM((L,), f32),
            pltpu.VMEM((L,), f32),
            pltpu.VMEM((L,), f32),
            pltpu.VMEM_SHARED((2 * POOL,), f32),
            pltpu.VMEM_SHARED((2 * POOL,), i32),
            pltpu.SemaphoreType.DMA,
        ],
    )(_sc_body)
    return kern(hm, rc, rs, ce, cz, dm)


def _epi_body(d_ref, box_ref, aux_ref):
    xs = (d_ref[:, 2] + d_ref[:, 4]) * 0.8 - 75.2
    ys = (d_ref[:, 3] + d_ref[:, 5]) * 0.8 - 75.2
    z = d_ref[:, 6]
    angle = jnp.arctan2(d_ref[:, 10], d_ref[:, 11])
    box_ref[0, :] = xs
    box_ref[1, :] = ys
    box_ref[2, :] = z
    box_ref[3, :] = d_ref[:, 7]
    box_ref[4, :] = d_ref[:, 8]
    box_ref[5, :] = d_ref[:, 9]
    box_ref[6, :] = angle
    box_ref[7, :] = angle
    inb = (xs >= -80.0) & (xs <= 80.0) & (ys >= -80.0) & (ys <= 80.0)
    inb = inb & (z >= -10.0) & (z <= 10.0)
    aux_ref[0, :] = d_ref[:, 0]
    aux_ref[1, :] = d_ref[:, 1]
    aux_ref[2, :] = inb.astype(jnp.float32)
    aux_ref[3, :] = angle


def _epilogue(outs):
    n = B * KP
    return pl.pallas_call(
        _epi_body,
        out_shape=(
            jax.ShapeDtypeStruct((8, n), jnp.float32),
            jax.ShapeDtypeStruct((8, n), jnp.float32),
        ),
    )(outs)


def kernel(heatmap, rot_cos, rot_sin, center, center_z, dim):
    hm = heatmap.reshape(-1)
    rc = rot_cos.reshape(-1)
    rs = rot_sin.reshape(-1)
    ce = center.reshape(-1)
    cz = center_z.reshape(-1)
    dm = dim.reshape(-1)
    outs = _sc_topk(hm, rc, rs, ce, cz, dm)
    box8, aux = _epilogue(outs.reshape(B * KP, L))
    box = jnp.transpose(box8.reshape(8, B, KP), (1, 2, 0))[:, :K, :7]
    score = aux[0].reshape(B, KP)[:, :K]
    cls = aux[1].reshape(B, KP)[:, :K].astype(jnp.int32)
    mask = aux[2].reshape(B, KP)[:, :K] > 0.5
    return box, score, cls, mask
